# baseline (speedup 1.0000x reference)
import functools

import jax
import jax.numpy as jnp
from jax import lax
from jax.experimental import pallas as pl
from jax.experimental.pallas import tpu as pltpu

AUX_COEF = 0.01
Z_COEF = 0.001
BLK = 1024


def _router_body(x_hbm, w_ref, i0_ref, i1_ref, w0_ref, w1_ref,
                 cnt_ref, ps_ref, lse_ref, xbuf, sem, *, n_experts, nb):
    i = pl.program_id(0)

    def copy_in(blk_idx, s):
        return pltpu.make_async_copy(
            x_hbm.at[pl.ds(blk_idx * BLK, BLK), :],
            xbuf.at[s],
            sem.at[s],
        )

    @pl.when(i < nb)
    def _start():
        copy_in(i, lax.rem(i, 2)).start()

    @pl.when(i > 0)
    def _compute():
        j = i - 1
        xblk = xbuf[lax.rem(j, 2)]
        logits = jnp.dot(xblk, w_ref[...],
                         preferred_element_type=jnp.float32)
        iota = jax.lax.broadcasted_iota(jnp.int32, logits.shape, 1)

        m0 = jnp.max(logits, axis=1, keepdims=True)
        i0 = jnp.min(jnp.where(logits == m0, iota, n_experts), axis=1,
                     keepdims=True)
        masked = jnp.where(iota == i0, jnp.float32(-1e30), logits)
        m1 = jnp.max(masked, axis=1, keepdims=True)
        i1 = jnp.min(jnp.where(masked == m1, iota, n_experts), axis=1,
                     keepdims=True)

        e1 = jnp.exp(m1 - m0)
        denom = 1.0 + e1
        w0_ref[...] = 1.0 / denom
        w1_ref[...] = e1 / denom
        i0_ref[...] = i0
        i1_ref[...] = i1

        ex = jnp.exp(logits - m0)
        ssum = jnp.sum(ex, axis=1, keepdims=True)
        ps_blk = jnp.sum(ex * (1.0 / ssum), axis=0)[None, :]
        lse_blk = jnp.sum(m0 + jnp.log(ssum), keepdims=True)
        one_hot = ((iota == i0).astype(jnp.float32)
                   + (iota == i1).astype(jnp.float32))
        cnt_blk = jnp.sum(one_hot, axis=0)[None, :]

        @pl.when(j == 0)
        def _init():
            cnt_ref[...] = jnp.zeros_like(cnt_ref)
            ps_ref[...] = jnp.zeros_like(ps_ref)
            lse_ref[...] = jnp.zeros_like(lse_ref)

        cnt_ref[...] += cnt_blk
        ps_ref[...] += ps_blk
        lse_ref[...] += lse_blk

    @pl.when(i < nb)
    def _wait():
        copy_in(i, lax.rem(i, 2)).wait()


def kernel(x, W):
    B, S, D = x.shape
    E = W.shape[1]
    N = B * S
    nb = N // BLK
    x2 = x.reshape(N, D)

    def out_idx(i):
        return (jnp.maximum(i - 1, 0), 0)

    body = functools.partial(_router_body, n_experts=E, nb=nb)
    i0, i1, w0, w1, cnt, ps, lse = pl.pallas_call(
        body,
        grid=(nb + 1,),
        in_specs=[
            pl.BlockSpec(memory_space=pl.ANY),
            pl.BlockSpec((D, E), lambda i: (0, 0)),
        ],
        out_specs=[
            pl.BlockSpec((BLK, 1), out_idx),
            pl.BlockSpec((BLK, 1), out_idx),
            pl.BlockSpec((BLK, 1), out_idx),
            pl.BlockSpec((BLK, 1), out_idx),
            pl.BlockSpec((1, E), lambda i: (0, 0)),
            pl.BlockSpec((1, E), lambda i: (0, 0)),
            pl.BlockSpec((1, 1), lambda i: (0, 0)),
        ],
        out_shape=[
            jax.ShapeDtypeStruct((N, 1), jnp.int32),
            jax.ShapeDtypeStruct((N, 1), jnp.int32),
            jax.ShapeDtypeStruct((N, 1), jnp.float32),
            jax.ShapeDtypeStruct((N, 1), jnp.float32),
            jax.ShapeDtypeStruct((1, E), jnp.float32),
            jax.ShapeDtypeStruct((1, E), jnp.float32),
            jax.ShapeDtypeStruct((1, 1), jnp.float32),
        ],
        scratch_shapes=[
            pltpu.VMEM((2, BLK, D), jnp.float32),
            pltpu.SemaphoreType.DMA((2,)),
        ],
    )(x2, W)

    idx = jnp.concatenate([i0, i1], axis=1).reshape(B, S, 2)
    wts = jnp.concatenate([w0, w1], axis=1).reshape(B, S, 2)
    tokens_per_expert = cnt[0] / N
    router_prob_per_expert = ps[0] / N
    balance_loss = jnp.sum(tokens_per_expert * router_prob_per_expert) * E
    z_loss = (lse[0, 0] / N) ** 2
    return (idx, wts, balance_loss * AUX_COEF, z_loss * Z_COEF,
            tokens_per_expert)


# speedup vs baseline: 1.0374x; 1.0374x over previous
import functools

import jax
import jax.numpy as jnp
from jax.experimental import pallas as pl
from jax.experimental.pallas import tpu as pltpu

AUX_COEF = 0.01
Z_COEF = 0.001
BLK = 1024


def _process(xblk, w_ref, iota_shape_e, half, i0_ref, i1_ref, w0_ref,
             w1_ref, cnt_ref, ps_ref, lse_ref, is_first):
    n_experts = iota_shape_e
    logits = jnp.dot(xblk, w_ref[...],
                     preferred_element_type=jnp.float32)
    iota = jax.lax.broadcasted_iota(jnp.int32, logits.shape, 1)

    m0 = jnp.max(logits, axis=1, keepdims=True)
    i0 = jnp.min(jnp.where(logits == m0, iota, n_experts), axis=1,
                 keepdims=True)
    masked = jnp.where(iota == i0, jnp.float32(-1e30), logits)
    m1 = jnp.max(masked, axis=1, keepdims=True)
    i1 = jnp.min(jnp.where(masked == m1, iota, n_experts), axis=1,
                 keepdims=True)

    e1 = jnp.exp(m1 - m0)
    denom = 1.0 + e1
    sl = pl.ds(half * BLK, BLK)
    w0_ref[sl, :] = 1.0 / denom
    w1_ref[sl, :] = e1 / denom
    i0_ref[sl, :] = i0
    i1_ref[sl, :] = i1

    ex = jnp.exp(logits - m0)
    ssum = jnp.sum(ex, axis=1, keepdims=True)
    ps_blk = jnp.sum(ex * (1.0 / ssum), axis=0)[None, :]
    lse_blk = jnp.sum(m0 + jnp.log(ssum), keepdims=True)
    one_hot = ((iota == i0).astype(jnp.float32)
               + (iota == i1).astype(jnp.float32))
    cnt_blk = jnp.sum(one_hot, axis=0)[None, :]

    @pl.when(is_first)
    def _init():
        cnt_ref[...] = jnp.zeros_like(cnt_ref)
        ps_ref[...] = jnp.zeros_like(ps_ref)
        lse_ref[...] = jnp.zeros_like(lse_ref)

    cnt_ref[...] += cnt_blk
    ps_ref[...] += ps_blk
    lse_ref[...] += lse_blk


def _router_body(x_hbm, w_ref, i0_ref, i1_ref, w0_ref, w1_ref,
                 cnt_ref, ps_ref, lse_ref, xb0, xb1, sem, *, n_experts,
                 nsteps):
    k = pl.program_id(0)

    def copy_in(blk_idx, buf, s):
        return pltpu.make_async_copy(
            x_hbm.at[pl.ds(blk_idx * BLK, BLK), :], buf, sem.at[s])

    @pl.when(k == 0)
    def _prime():
        copy_in(0, xb0, 0).start()
        copy_in(1, xb1, 1).start()

    outs = (i0_ref, i1_ref, w0_ref, w1_ref, cnt_ref, ps_ref, lse_ref)

    copy_in(2 * k, xb0, 0).wait()
    _process(xb0[...], w_ref, n_experts, 0, *outs, is_first=(k == 0))

    @pl.when(k + 1 < nsteps)
    def _refill0():
        copy_in(2 * k + 2, xb0, 0).start()

    copy_in(2 * k + 1, xb1, 1).wait()
    _process(xb1[...], w_ref, n_experts, 1, *outs,
             is_first=jnp.bool_(False))

    @pl.when(k + 1 < nsteps)
    def _refill1():
        copy_in(2 * k + 3, xb1, 1).start()


def kernel(x, W):
    B, S, D = x.shape
    E = W.shape[1]
    N = B * S
    nsteps = N // (2 * BLK)
    x2 = x.reshape(N, D)

    body = functools.partial(_router_body, n_experts=E, nsteps=nsteps)
    i0, i1, w0, w1, cnt, ps, lse = pl.pallas_call(
        body,
        grid=(nsteps,),
        in_specs=[
            pl.BlockSpec(memory_space=pl.ANY),
            pl.BlockSpec((D, E), lambda i: (0, 0)),
        ],
        out_specs=[
            pl.BlockSpec((2 * BLK, 1), lambda i: (i, 0)),
            pl.BlockSpec((2 * BLK, 1), lambda i: (i, 0)),
            pl.BlockSpec((2 * BLK, 1), lambda i: (i, 0)),
            pl.BlockSpec((2 * BLK, 1), lambda i: (i, 0)),
            pl.BlockSpec((1, E), lambda i: (0, 0)),
            pl.BlockSpec((1, E), lambda i: (0, 0)),
            pl.BlockSpec((1, 1), lambda i: (0, 0)),
        ],
        out_shape=[
            jax.ShapeDtypeStruct((N, 1), jnp.int32),
            jax.ShapeDtypeStruct((N, 1), jnp.int32),
            jax.ShapeDtypeStruct((N, 1), jnp.float32),
            jax.ShapeDtypeStruct((N, 1), jnp.float32),
            jax.ShapeDtypeStruct((1, E), jnp.float32),
            jax.ShapeDtypeStruct((1, E), jnp.float32),
            jax.ShapeDtypeStruct((1, 1), jnp.float32),
        ],
        scratch_shapes=[
            pltpu.VMEM((BLK, D), jnp.float32),
            pltpu.VMEM((BLK, D), jnp.float32),
            pltpu.SemaphoreType.DMA((2,)),
        ],
    )(x2, W)

    idx = jnp.concatenate([i0, i1], axis=1).reshape(B, S, 2)
    wts = jnp.concatenate([w0, w1], axis=1).reshape(B, S, 2)
    tokens_per_expert = cnt[0] / N
    router_prob_per_expert = ps[0] / N
    balance_loss = jnp.sum(tokens_per_expert * router_prob_per_expert) * E
    z_loss = (lse[0, 0] / N) ** 2
    return (idx, wts, balance_loss * AUX_COEF, z_loss * Z_COEF,
            tokens_per_expert)


# speedup vs baseline: 1.8065x; 1.7415x over previous
import functools

import jax
import jax.numpy as jnp
from jax.experimental import pallas as pl
from jax.experimental.pallas import tpu as pltpu

AUX_COEF = 0.01
Z_COEF = 0.001
BLK = 2048


def _router_body(x_ref, w_ref, i0_ref, i1_ref, w0_ref, w1_ref,
                 cnt_ref, ps_ref, lse_ref, ps_acc, cnt_acc, lse_acc,
                 *, n_experts, nb):
    i = pl.program_id(0)
    logits = jnp.dot(x_ref[...], w_ref[...],
                     preferred_element_type=jnp.float32)
    lt = logits.T
    iota = jax.lax.broadcasted_iota(jnp.int32, lt.shape, 0)

    m0 = jnp.max(lt, axis=0, keepdims=True)
    i0 = jnp.min(jnp.where(lt == m0, iota, n_experts), axis=0,
                 keepdims=True)
    masked = jnp.where(iota == i0, jnp.float32(-1e30), lt)
    m1 = jnp.max(masked, axis=0, keepdims=True)
    i1 = jnp.min(jnp.where(masked == m1, iota, n_experts), axis=0,
                 keepdims=True)

    e1 = jnp.exp(m1 - m0)
    denom = 1.0 + e1
    w0_ref[...] = (1.0 / denom)[None]
    w1_ref[...] = (e1 / denom)[None]
    i0_ref[...] = i0[None]
    i1_ref[...] = i1[None]

    ex = jnp.exp(lt - m0)
    ssum = jnp.sum(ex, axis=0, keepdims=True)
    probs = ex * (1.0 / ssum)
    one_hot = ((iota == i0).astype(jnp.float32)
               + (iota == i1).astype(jnp.float32))
    lse_row = m0 + jnp.log(ssum)

    @pl.when(i == 0)
    def _init():
        ps_acc[...] = probs
        cnt_acc[...] = one_hot
        lse_acc[...] = lse_row

    @pl.when(i > 0)
    def _accum():
        ps_acc[...] += probs
        cnt_acc[...] += one_hot
        lse_acc[...] += lse_row

    @pl.when(i == nb - 1)
    def _finish():
        cnt_ref[...] = jnp.sum(cnt_acc[...], axis=1, keepdims=True)
        ps_ref[...] = jnp.sum(ps_acc[...], axis=1, keepdims=True)
        lse_ref[...] = jnp.sum(lse_acc[...], axis=1, keepdims=True)


def kernel(x, W):
    B, S, D = x.shape
    E = W.shape[1]
    N = B * S
    nb = N // BLK
    x2 = x.reshape(N, D)

    body = functools.partial(_router_body, n_experts=E, nb=nb)
    i0, i1, w0, w1, cnt, ps, lse = pl.pallas_call(
        body,
        grid=(nb,),
        in_specs=[
            pl.BlockSpec((BLK, D), lambda i: (i, 0)),
            pl.BlockSpec((D, E), lambda i: (0, 0)),
        ],
        out_specs=[
            pl.BlockSpec((1, 1, BLK), lambda i: (i, 0, 0)),
            pl.BlockSpec((1, 1, BLK), lambda i: (i, 0, 0)),
            pl.BlockSpec((1, 1, BLK), lambda i: (i, 0, 0)),
            pl.BlockSpec((1, 1, BLK), lambda i: (i, 0, 0)),
            pl.BlockSpec((E, 1), lambda i: (0, 0)),
            pl.BlockSpec((E, 1), lambda i: (0, 0)),
            pl.BlockSpec((1, 1), lambda i: (0, 0)),
        ],
        out_shape=[
            jax.ShapeDtypeStruct((nb, 1, BLK), jnp.int32),
            jax.ShapeDtypeStruct((nb, 1, BLK), jnp.int32),
            jax.ShapeDtypeStruct((nb, 1, BLK), jnp.float32),
            jax.ShapeDtypeStruct((nb, 1, BLK), jnp.float32),
            jax.ShapeDtypeStruct((E, 1), jnp.float32),
            jax.ShapeDtypeStruct((E, 1), jnp.float32),
            jax.ShapeDtypeStruct((1, 1), jnp.float32),
        ],
        scratch_shapes=[
            pltpu.VMEM((E, BLK), jnp.float32),
            pltpu.VMEM((E, BLK), jnp.float32),
            pltpu.VMEM((1, BLK), jnp.float32),
        ],
    )(x2, W)

    idx = jnp.stack([i0.reshape(N), i1.reshape(N)], axis=-1).reshape(B, S, 2)
    wts = jnp.stack([w0.reshape(N), w1.reshape(N)], axis=-1).reshape(B, S, 2)
    tokens_per_expert = cnt[:, 0] / N
    router_prob_per_expert = ps[:, 0] / N
    balance_loss = jnp.sum(tokens_per_expert * router_prob_per_expert) * E
    z_loss = (lse[0, 0] / N) ** 2
    return (idx, wts, balance_loss * AUX_COEF, z_loss * Z_COEF,
            tokens_per_expert)


# speedup vs baseline: 1.8219x; 1.0085x over previous
import functools

import jax
import jax.numpy as jnp
from jax.experimental import pallas as pl
from jax.experimental.pallas import tpu as pltpu

AUX_COEF = 0.01
Z_COEF = 0.001
BLK = 1024


def _router_body(x_ref, w_ref, i0_ref, i1_ref, w0_ref, w1_ref,
                 cnt_ref, ps_ref, lse_ref, ps_acc, cnt_acc, lse_acc,
                 *, n_experts, nb):
    i = pl.program_id(0)
    logits = jnp.dot(x_ref[...], w_ref[...],
                     preferred_element_type=jnp.float32)
    lt = logits.T
    iota = jax.lax.broadcasted_iota(jnp.int32, lt.shape, 0)

    m0 = jnp.max(lt, axis=0, keepdims=True)
    i0 = jnp.min(jnp.where(lt == m0, iota, n_experts), axis=0,
                 keepdims=True)
    masked = jnp.where(iota == i0, jnp.float32(-1e30), lt)
    m1 = jnp.max(masked, axis=0, keepdims=True)
    i1 = jnp.min(jnp.where(masked == m1, iota, n_experts), axis=0,
                 keepdims=True)

    e1 = jnp.exp(m1 - m0)
    denom = 1.0 + e1
    w0_ref[...] = (1.0 / denom)[None]
    w1_ref[...] = (e1 / denom)[None]
    i0_ref[...] = i0[None]
    i1_ref[...] = i1[None]

    ex = jnp.exp(lt - m0)
    ssum = jnp.sum(ex, axis=0, keepdims=True)
    probs = ex * (1.0 / ssum)
    one_hot = ((iota == i0).astype(jnp.float32)
               + (iota == i1).astype(jnp.float32))
    lse_row = m0 + jnp.log(ssum)

    @pl.when(i == 0)
    def _init():
        ps_acc[...] = probs
        cnt_acc[...] = one_hot
        lse_acc[...] = lse_row

    @pl.when(i > 0)
    def _accum():
        ps_acc[...] += probs
        cnt_acc[...] += one_hot
        lse_acc[...] += lse_row

    @pl.when(i == nb - 1)
    def _finish():
        cnt_ref[...] = jnp.sum(cnt_acc[...], axis=1, keepdims=True)
        ps_ref[...] = jnp.sum(ps_acc[...], axis=1, keepdims=True)
        lse_ref[...] = jnp.sum(lse_acc[...], axis=1, keepdims=True)


def kernel(x, W):
    B, S, D = x.shape
    E = W.shape[1]
    N = B * S
    nb = N // BLK
    x2 = x.reshape(N, D)

    body = functools.partial(_router_body, n_experts=E, nb=nb)
    i0, i1, w0, w1, cnt, ps, lse = pl.pallas_call(
        body,
        grid=(nb,),
        in_specs=[
            pl.BlockSpec((BLK, D), lambda i: (i, 0)),
            pl.BlockSpec((D, E), lambda i: (0, 0)),
        ],
        out_specs=[
            pl.BlockSpec((1, 1, BLK), lambda i: (i, 0, 0)),
            pl.BlockSpec((1, 1, BLK), lambda i: (i, 0, 0)),
            pl.BlockSpec((1, 1, BLK), lambda i: (i, 0, 0)),
            pl.BlockSpec((1, 1, BLK), lambda i: (i, 0, 0)),
            pl.BlockSpec((E, 1), lambda i: (0, 0)),
            pl.BlockSpec((E, 1), lambda i: (0, 0)),
            pl.BlockSpec((1, 1), lambda i: (0, 0)),
        ],
        out_shape=[
            jax.ShapeDtypeStruct((nb, 1, BLK), jnp.int32),
            jax.ShapeDtypeStruct((nb, 1, BLK), jnp.int32),
            jax.ShapeDtypeStruct((nb, 1, BLK), jnp.float32),
            jax.ShapeDtypeStruct((nb, 1, BLK), jnp.float32),
            jax.ShapeDtypeStruct((E, 1), jnp.float32),
            jax.ShapeDtypeStruct((E, 1), jnp.float32),
            jax.ShapeDtypeStruct((1, 1), jnp.float32),
        ],
        scratch_shapes=[
            pltpu.VMEM((E, BLK), jnp.float32),
            pltpu.VMEM((E, BLK), jnp.float32),
            pltpu.VMEM((1, BLK), jnp.float32),
        ],
    )(x2, W)

    idx = jnp.stack([i0.reshape(N), i1.reshape(N)], axis=-1).reshape(B, S, 2)
    wts = jnp.stack([w0.reshape(N), w1.reshape(N)], axis=-1).reshape(B, S, 2)
    tokens_per_expert = cnt[:, 0] / N
    router_prob_per_expert = ps[:, 0] / N
    balance_loss = jnp.sum(tokens_per_expert * router_prob_per_expert) * E
    z_loss = (lse[0, 0] / N) ** 2
    return (idx, wts, balance_loss * AUX_COEF, z_loss * Z_COEF,
            tokens_per_expert)
